# Initial kernel scaffold; baseline (speedup 1.0000x reference)
import functools

import jax
import jax.numpy as jnp
import numpy as np
from jax import lax
from jax.experimental import pallas as pl
from jax.experimental.pallas import tpu as pltpu
from jax.experimental.pallas import tpu_sc as plsc

NN = 10000
EE = 320000
HH = 128
CHUNK = 128
NCHUNKS = EE // CHUNK
NCORES = 2
NSUB = 16
NTILES = NCORES * NSUB
ROWS_PER_TILE = NN // NSUB
RB = 1000
NBLK = NN // RB
EPSV = 1e-5

_F32 = jnp.float32


def _vmesh():
    return plsc.VectorSubcoreMesh(core_axis_name="c", subcore_axis_name="s")


def _sc_degrees(edge_index, zvec):
    @functools.partial(
        pl.kernel,
        out_type=jax.ShapeDtypeStruct((NCORES, NN), _F32),
        mesh=_vmesh(),
        scratch_types=[
            pltpu.VMEM_SHARED((NN,), _F32),
            pltpu.VMEM((CHUNK,), jnp.int32),
            pltpu.VMEM((CHUNK,), _F32),
        ],
    )
    def k(ei_hbm, z_hbm, out_hbm, acc_sh, idx_v, ones_v):
        c = lax.axis_index("c")
        s = lax.axis_index("s")

        @pl.loop(0, CHUNK // 16)
        def _(j):
            ones_v[pl.ds(j * 16, 16)] = jnp.full((16,), 1.0, _F32)

        @pl.when(s == 0)
        def _():
            pltpu.sync_copy(z_hbm, acc_sh)

        plsc.subcore_barrier()

        @pl.loop(0, (NCHUNKS + NSUB - 1) // NSUB)
        def _(i):
            j = s + i * NSUB

            @pl.when(j < NCHUNKS)
            def _():
                pltpu.sync_copy(ei_hbm.at[c].at[pl.ds(j * CHUNK, CHUNK)], idx_v)
                pltpu.sync_copy(ones_v, acc_sh.at[idx_v], add=True)

        plsc.subcore_barrier()

        @pl.when(s == 0)
        def _():
            pltpu.sync_copy(acc_sh, out_hbm.at[c])

    return k(edge_index, zvec)


def _sc_aggregate(m, src, dst, zrows):
    @functools.partial(
        pl.kernel,
        out_type=jax.ShapeDtypeStruct((NCORES, NN, HH), _F32),
        mesh=_vmesh(),
        scratch_types=[
            pltpu.VMEM_SHARED((NN, HH), _F32),
            pltpu.VMEM((CHUNK,), jnp.int32),
            pltpu.VMEM((CHUNK,), jnp.int32),
            pltpu.VMEM((CHUNK, HH), _F32),
        ],
    )
    def k(m_hbm, src_hbm, dst_hbm, z_hbm, out_hbm, acc_sh, si_v, di_v, rows_v):
        c = lax.axis_index("c")
        s = lax.axis_index("s")
        tid = c * NSUB + s

        r0 = s * ROWS_PER_TILE
        pltpu.sync_copy(
            z_hbm.at[pl.ds(r0, ROWS_PER_TILE)],
            acc_sh.at[pl.ds(r0, ROWS_PER_TILE)],
        )
        plsc.subcore_barrier()

        @pl.loop(0, (NCHUNKS + NTILES - 1) // NTILES)
        def _(i):
            j = tid + i * NTILES

            @pl.when(j < NCHUNKS)
            def _():
                base = j * CHUNK
                pltpu.sync_copy(src_hbm.at[pl.ds(base, CHUNK)], si_v)
                pltpu.sync_copy(dst_hbm.at[pl.ds(base, CHUNK)], di_v)
                pltpu.sync_copy(m_hbm.at[si_v], rows_v)
                pltpu.sync_copy(rows_v, acc_sh.at[di_v], add=True)

        plsc.subcore_barrier()

        pltpu.sync_copy(
            acc_sh.at[pl.ds(r0, ROWS_PER_TILE)],
            out_hbm.at[c].at[pl.ds(r0, ROWS_PER_TILE)],
        )

    return k(m, src, dst, zrows)


def _tc_norms(deg2):
    def body(d_ref, o_ref):
        o_ref[...] = lax.rsqrt(jnp.maximum(d_ref[...], 1.0))

    return pl.pallas_call(
        body,
        out_shape=jax.ShapeDtypeStruct((NCORES, NN), _F32),
    )(deg2)


def _tc_layer0(x, ns, W):
    def body(x_ref, ns_ref, w_ref, o_ref):
        o_ref[...] = jnp.dot(
            x_ref[...] * ns_ref[...], w_ref[...], preferred_element_type=_F32
        )

    return pl.pallas_call(
        body,
        grid=(NBLK,),
        in_specs=[
            pl.BlockSpec((RB, HH), lambda i: (i, 0)),
            pl.BlockSpec((RB, 1), lambda i: (i, 0)),
            pl.BlockSpec((HH, HH), lambda i: (0, 0)),
        ],
        out_specs=pl.BlockSpec((RB, HH), lambda i: (i, 0)),
        out_shape=jax.ShapeDtypeStruct((NN, HH), _F32),
    )(x, ns, W)


def _tc_mid(p, nd, b, ns, W):
    def body(p_ref, nd_ref, b_ref, ns_ref, w_ref, o_ref):
        pr = p_ref[...]
        h = jnp.maximum((pr[0] + pr[1]) * nd_ref[...] + b_ref[...], 0.0)
        o_ref[...] = jnp.dot(
            h * ns_ref[...], w_ref[...], preferred_element_type=_F32
        )

    return pl.pallas_call(
        body,
        grid=(NBLK,),
        in_specs=[
            pl.BlockSpec((NCORES, RB, HH), lambda i: (0, i, 0)),
            pl.BlockSpec((RB, 1), lambda i: (i, 0)),
            pl.BlockSpec((1, HH), lambda i: (0, 0)),
            pl.BlockSpec((RB, 1), lambda i: (i, 0)),
            pl.BlockSpec((HH, HH), lambda i: (0, 0)),
        ],
        out_specs=pl.BlockSpec((RB, HH), lambda i: (i, 0)),
        out_shape=jax.ShapeDtypeStruct((NN, HH), _F32),
    )(p, nd, b, ns, W)


def _tc_final(p, nd, b, w1, b1, g, be, w2, b2):
    def body(p_ref, nd_ref, b_ref, w1_ref, b1_ref, g_ref, be_ref, w2_ref,
             b2_ref, o_ref, sacc, macc):
        i = pl.program_id(0)
        pr = p_ref[...]
        h = jnp.maximum((pr[0] + pr[1]) * nd_ref[...] + b_ref[...], 0.0)
        bs = jnp.sum(h, axis=0, keepdims=True)
        bm = jnp.max(h, axis=0, keepdims=True)

        @pl.when(i == 0)
        def _():
            sacc[...] = bs
            macc[...] = bm

        @pl.when(i > 0)
        def _():
            sacc[...] += bs
            macc[...] = jnp.maximum(macc[...], bm)

        @pl.when(i == NBLK - 1)
        def _():
            rs = sacc[...]
            rm = rs * (1.0 / NN)
            rx = macc[...]
            w1r = w1_ref[...]
            z = (
                jnp.dot(rs, w1r[0:HH], preferred_element_type=_F32)
                + jnp.dot(rm, w1r[HH:2 * HH], preferred_element_type=_F32)
                + jnp.dot(rx, w1r[2 * HH:3 * HH], preferred_element_type=_F32)
                + b1_ref[...]
            )
            z = z * (g_ref[...] * float(1.0 / np.sqrt(1.0 + EPSV))) + be_ref[...]
            z = jnp.maximum(z, 0.0)
            o_ref[...] = (
                jnp.dot(z, w2_ref[...], preferred_element_type=_F32)
                + b2_ref[...]
            )

    return pl.pallas_call(
        body,
        grid=(NBLK,),
        in_specs=[
            pl.BlockSpec((NCORES, RB, HH), lambda i: (0, i, 0)),
            pl.BlockSpec((RB, 1), lambda i: (i, 0)),
            pl.BlockSpec((1, HH), lambda i: (0, 0)),
            pl.BlockSpec((3 * HH, HH), lambda i: (0, 0)),
            pl.BlockSpec((1, HH), lambda i: (0, 0)),
            pl.BlockSpec((1, HH), lambda i: (0, 0)),
            pl.BlockSpec((1, HH), lambda i: (0, 0)),
            pl.BlockSpec((HH, 1), lambda i: (0, 0)),
            pl.BlockSpec((1, 1), lambda i: (0, 0)),
        ],
        out_specs=pl.BlockSpec((1, 1), lambda i: (0, 0)),
        out_shape=jax.ShapeDtypeStruct((1, 1), _F32),
        scratch_shapes=[
            pltpu.VMEM((1, HH), _F32),
            pltpu.VMEM((1, HH), _F32),
        ],
    )(p, nd, b, w1, b1, g, be, w2, b2)


def kernel(x, edge_index, W0, b0, W1, b1, W2, b2, W3, b3, W4, b4,
           mlpW1, mlpb1, gamma, beta, mlpW2, mlpb2):
    zvec = jnp.zeros((NN,), _F32)
    zrows = jnp.zeros((NN, HH), _F32)
    src = edge_index[0]
    dst = edge_index[1]

    deg2 = _sc_degrees(edge_index, zvec)
    norm2 = _tc_norms(deg2)
    ns = norm2[0].reshape(NN, 1)
    nd = norm2[1].reshape(NN, 1)

    Ws = [W0, W1, W2, W3, W4]
    bs = [b0.reshape(1, HH), b1.reshape(1, HH), b2.reshape(1, HH),
          b3.reshape(1, HH), b4.reshape(1, HH)]

    m = _tc_layer0(x, ns, Ws[0])
    p = None
    for l in range(5):
        p = _sc_aggregate(m, src, dst, zrows)
        if l < 4:
            m = _tc_mid(p, nd, bs[l], ns, Ws[l + 1])

    return _tc_final(
        p, nd, bs[4], mlpW1, mlpb1.reshape(1, HH), gamma.reshape(1, HH),
        beta.reshape(1, HH), mlpW2, mlpb2.reshape(1, 1),
    )


# speedup vs baseline: 5.9558x; 5.9558x over previous
import functools

import jax
import jax.numpy as jnp
import numpy as np
from jax import lax
from jax.experimental import pallas as pl
from jax.experimental.pallas import tpu as pltpu
from jax.experimental.pallas import tpu_sc as plsc

NN = 10000
EE = 320000
HH = 128
CHUNK = 128
NCHUNKS = EE // CHUNK
NCORES = 2
NSUB = 16
NTILES = NCORES * NSUB
ROWS_PER_TILE = 624
TAIL_ROW0 = ROWS_PER_TILE * NSUB
TAIL_ROWS = NN - TAIL_ROW0
RB = 1000
NBLK = NN // RB
EPSV = 1e-5

_F32 = jnp.float32


def _vmesh():
    return plsc.VectorSubcoreMesh(core_axis_name="c", subcore_axis_name="s")


def _sc_degrees(src, dst, zvec):
    @functools.partial(
        pl.kernel,
        out_type=[jax.ShapeDtypeStruct((NN,), _F32),
                  jax.ShapeDtypeStruct((NN,), _F32)],
        mesh=_vmesh(),
        scratch_types=[
            pltpu.VMEM_SHARED((NN,), _F32),
            pltpu.VMEM((CHUNK,), jnp.int32),
            pltpu.VMEM((CHUNK,), _F32),
        ],
    )
    def k(src_hbm, dst_hbm, z_hbm, od_hbm, id_hbm, acc_sh, idx_v, ones_v):
        c = lax.axis_index("c")
        s = lax.axis_index("s")

        @pl.loop(0, CHUNK // 16)
        def _(j):
            ones_v[pl.ds(j * 16, 16)] = jnp.full((16,), 1.0, _F32)

        @pl.when(s == 0)
        def _():
            pltpu.sync_copy(z_hbm, acc_sh)

        plsc.subcore_barrier()

        @pl.loop(0, (NCHUNKS + NSUB - 1) // NSUB)
        def _(i):
            j = s + i * NSUB

            @pl.when(j < NCHUNKS)
            def _():
                @pl.when(c == 0)
                def _():
                    pltpu.sync_copy(src_hbm.at[pl.ds(j * CHUNK, CHUNK)], idx_v)

                @pl.when(c == 1)
                def _():
                    pltpu.sync_copy(dst_hbm.at[pl.ds(j * CHUNK, CHUNK)], idx_v)

                pltpu.sync_copy(ones_v, acc_sh.at[idx_v], add=True)

        plsc.subcore_barrier()

        @pl.when(s == 0)
        def _():
            @pl.when(c == 0)
            def _():
                pltpu.sync_copy(acc_sh, od_hbm)

            @pl.when(c == 1)
            def _():
                pltpu.sync_copy(acc_sh, id_hbm)

    return k(src, dst, zvec)


def _sc_aggregate(m, src, dst, zrows):
    @functools.partial(
        pl.kernel,
        out_type=jax.ShapeDtypeStruct((NCORES, NN, HH), _F32),
        mesh=_vmesh(),
        scratch_types=[
            pltpu.VMEM_SHARED((NN, HH), _F32),
            pltpu.VMEM((CHUNK,), jnp.int32),
            pltpu.VMEM((CHUNK,), jnp.int32),
            pltpu.VMEM((CHUNK, HH), _F32),
        ],
    )
    def k(m_hbm, src_hbm, dst_hbm, z_hbm, out_hbm, acc_sh, si_v, di_v, rows_v):
        c = lax.axis_index("c")
        s = lax.axis_index("s")
        tid = c * NSUB + s

        r0 = s * ROWS_PER_TILE
        pltpu.sync_copy(
            z_hbm.at[pl.ds(r0, ROWS_PER_TILE)],
            acc_sh.at[pl.ds(r0, ROWS_PER_TILE)],
        )

        @pl.when(s == NSUB - 1)
        def _():
            pltpu.sync_copy(
                z_hbm.at[pl.ds(TAIL_ROW0, TAIL_ROWS)],
                acc_sh.at[pl.ds(TAIL_ROW0, TAIL_ROWS)],
            )

        plsc.subcore_barrier()

        @pl.loop(0, (NCHUNKS + NTILES - 1) // NTILES)
        def _(i):
            j = tid + i * NTILES

            @pl.when(j < NCHUNKS)
            def _():
                base = j * CHUNK
                pltpu.sync_copy(src_hbm.at[pl.ds(base, CHUNK)], si_v)
                pltpu.sync_copy(dst_hbm.at[pl.ds(base, CHUNK)], di_v)
                pltpu.sync_copy(m_hbm.at[si_v], rows_v)
                pltpu.sync_copy(rows_v, acc_sh.at[di_v], add=True)

        plsc.subcore_barrier()

        pltpu.sync_copy(
            acc_sh.at[pl.ds(r0, ROWS_PER_TILE)],
            out_hbm.at[c].at[pl.ds(r0, ROWS_PER_TILE)],
        )

        @pl.when(s == NSUB - 1)
        def _():
            pltpu.sync_copy(
                acc_sh.at[pl.ds(TAIL_ROW0, TAIL_ROWS)],
                out_hbm.at[c].at[pl.ds(TAIL_ROW0, TAIL_ROWS)],
            )

    return k(m, src, dst, zrows)


def _tc_norms(od, idg):
    def body(od_ref, id_ref, ns_ref, nd_ref):
        ns_ref[...] = lax.rsqrt(jnp.maximum(od_ref[...], 1.0))
        nd_ref[...] = lax.rsqrt(jnp.maximum(id_ref[...], 1.0))

    return pl.pallas_call(
        body,
        out_shape=[jax.ShapeDtypeStruct((NN,), _F32),
                   jax.ShapeDtypeStruct((NN,), _F32)],
    )(od, idg)


def _tc_layer0(x, ns, W):
    def body(x_ref, ns_ref, w_ref, o_ref):
        o_ref[...] = jnp.dot(
            x_ref[...] * ns_ref[...], w_ref[...], preferred_element_type=_F32
        )

    return pl.pallas_call(
        body,
        grid=(NBLK,),
        in_specs=[
            pl.BlockSpec((RB, HH), lambda i: (i, 0)),
            pl.BlockSpec((RB, 1), lambda i: (i, 0)),
            pl.BlockSpec((HH, HH), lambda i: (0, 0)),
        ],
        out_specs=pl.BlockSpec((RB, HH), lambda i: (i, 0)),
        out_shape=jax.ShapeDtypeStruct((NN, HH), _F32),
    )(x, ns, W)


def _tc_mid(p, nd, b, ns, W):
    def body(p_ref, nd_ref, b_ref, ns_ref, w_ref, o_ref):
        pr = p_ref[...]
        h = jnp.maximum((pr[0] + pr[1]) * nd_ref[...] + b_ref[...], 0.0)
        o_ref[...] = jnp.dot(
            h * ns_ref[...], w_ref[...], preferred_element_type=_F32
        )

    return pl.pallas_call(
        body,
        grid=(NBLK,),
        in_specs=[
            pl.BlockSpec((NCORES, RB, HH), lambda i: (0, i, 0)),
            pl.BlockSpec((RB, 1), lambda i: (i, 0)),
            pl.BlockSpec((1, HH), lambda i: (0, 0)),
            pl.BlockSpec((RB, 1), lambda i: (i, 0)),
            pl.BlockSpec((HH, HH), lambda i: (0, 0)),
        ],
        out_specs=pl.BlockSpec((RB, HH), lambda i: (i, 0)),
        out_shape=jax.ShapeDtypeStruct((NN, HH), _F32),
    )(p, nd, b, ns, W)


def _tc_final(p, nd, b, w1, b1, g, be, w2, b2):
    def body(p_ref, nd_ref, b_ref, w1_ref, b1_ref, g_ref, be_ref, w2_ref,
             b2_ref, o_ref, sacc, macc):
        i = pl.program_id(0)
        pr = p_ref[...]
        h = jnp.maximum((pr[0] + pr[1]) * nd_ref[...] + b_ref[...], 0.0)
        bs = jnp.sum(h, axis=0, keepdims=True)
        bm = jnp.max(h, axis=0, keepdims=True)

        @pl.when(i == 0)
        def _():
            sacc[...] = bs
            macc[...] = bm

        @pl.when(i > 0)
        def _():
            sacc[...] += bs
            macc[...] = jnp.maximum(macc[...], bm)

        @pl.when(i == NBLK - 1)
        def _():
            rs = sacc[...]
            rm = rs * (1.0 / NN)
            rx = macc[...]
            w1r = w1_ref[...]
            z = (
                jnp.dot(rs, w1r[0:HH], preferred_element_type=_F32)
                + jnp.dot(rm, w1r[HH:2 * HH], preferred_element_type=_F32)
                + jnp.dot(rx, w1r[2 * HH:3 * HH], preferred_element_type=_F32)
                + b1_ref[...]
            )
            z = z * (g_ref[...] * float(1.0 / np.sqrt(1.0 + EPSV))) + be_ref[...]
            z = jnp.maximum(z, 0.0)
            o_ref[...] = (
                jnp.dot(z, w2_ref[...], preferred_element_type=_F32)
                + b2_ref[...]
            )

    return pl.pallas_call(
        body,
        grid=(NBLK,),
        in_specs=[
            pl.BlockSpec((NCORES, RB, HH), lambda i: (0, i, 0)),
            pl.BlockSpec((RB, 1), lambda i: (i, 0)),
            pl.BlockSpec((1, HH), lambda i: (0, 0)),
            pl.BlockSpec((3 * HH, HH), lambda i: (0, 0)),
            pl.BlockSpec((1, HH), lambda i: (0, 0)),
            pl.BlockSpec((1, HH), lambda i: (0, 0)),
            pl.BlockSpec((1, HH), lambda i: (0, 0)),
            pl.BlockSpec((HH, 1), lambda i: (0, 0)),
            pl.BlockSpec((1, 1), lambda i: (0, 0)),
        ],
        out_specs=pl.BlockSpec((1, 1), lambda i: (0, 0)),
        out_shape=jax.ShapeDtypeStruct((1, 1), _F32),
        scratch_shapes=[
            pltpu.VMEM((1, HH), _F32),
            pltpu.VMEM((1, HH), _F32),
        ],
    )(p, nd, b, w1, b1, g, be, w2, b2)


def kernel(x, edge_index, W0, b0, W1, b1, W2, b2, W3, b3, W4, b4,
           mlpW1, mlpb1, gamma, beta, mlpW2, mlpb2):
    zvec = jnp.zeros((NN,), _F32)
    zrows = jnp.zeros((NN, HH), _F32)
    src = edge_index[0]
    dst = edge_index[1]

    od, idg = _sc_degrees(src, dst, zvec)
    ns, nd = _tc_norms(od, idg)
    ns = ns.reshape(NN, 1)
    nd = nd.reshape(NN, 1)

    Ws = [W0, W1, W2, W3, W4]
    bs = [b0.reshape(1, HH), b1.reshape(1, HH), b2.reshape(1, HH),
          b3.reshape(1, HH), b4.reshape(1, HH)]

    m = _tc_layer0(x, ns, Ws[0])
    p = None
    for l in range(5):
        p = _sc_aggregate(m, src, dst, zrows)
        if l < 4:
            m = _tc_mid(p, nd, bs[l], ns, Ws[l + 1])

    return _tc_final(
        p, nd, bs[4], mlpW1, mlpb1.reshape(1, HH), gamma.reshape(1, HH),
        beta.reshape(1, HH), mlpW2, mlpb2.reshape(1, 1),
    )


# speedup vs baseline: 12.2729x; 2.0607x over previous
import functools

import jax
import jax.numpy as jnp
import numpy as np
from jax import lax
from jax.experimental import pallas as pl
from jax.experimental.pallas import tpu as pltpu
from jax.experimental.pallas import tpu_sc as plsc

NN = 10000
EE = 320000
HH = 128
CHUNK = 128
NCHUNKS = EE // CHUNK
NCORES = 2
NSUB = 16
NTILES = NCORES * NSUB
ROWS_PER_TILE = 624
TAIL_ROW0 = ROWS_PER_TILE * NSUB
TAIL_ROWS = NN - TAIL_ROW0
SPAN = 80
HSPAN = SPAN // 2
NCHP = SPAN * NTILES
DSPAN = NCHP // NSUB
RB = 1000
NBLK = NN // RB
EPSV = 1e-5

_F32 = jnp.float32


def _vmesh():
    return plsc.VectorSubcoreMesh(core_axis_name="c", subcore_axis_name="s")


def _sc_degrees(src2d, dst2d, zvec):
    @functools.partial(
        pl.kernel,
        out_type=[jax.ShapeDtypeStruct((NN,), _F32),
                  jax.ShapeDtypeStruct((NN,), _F32)],
        mesh=_vmesh(),
        scratch_types=[
            pltpu.VMEM_SHARED((NN,), _F32),
            pltpu.VMEM((DSPAN, CHUNK), jnp.int32),
            pltpu.VMEM((CHUNK,), _F32),
        ],
    )
    def k(src_hbm, dst_hbm, z_hbm, od_hbm, id_hbm, acc_sh, idx_v, ones_v):
        c = lax.axis_index("c")
        s = lax.axis_index("s")
        j0 = s * DSPAN

        @pl.loop(0, CHUNK // 16)
        def _(j):
            ones_v[pl.ds(j * 16, 16)] = jnp.full((16,), 1.0, _F32)

        @pl.when(c == 0)
        def _():
            pltpu.sync_copy(src_hbm.at[pl.ds(j0, DSPAN)], idx_v)

        @pl.when(c == 1)
        def _():
            pltpu.sync_copy(dst_hbm.at[pl.ds(j0, DSPAN)], idx_v)

        @pl.when(s == 0)
        def _():
            pltpu.sync_copy(z_hbm, acc_sh)

        plsc.subcore_barrier()

        @pl.loop(0, DSPAN)
        def _(i):
            @pl.when(j0 + i < NCHUNKS)
            def _():
                pltpu.sync_copy(ones_v, acc_sh.at[idx_v.at[i]], add=True)

        plsc.subcore_barrier()

        @pl.when(s == 0)
        def _():
            @pl.when(c == 0)
            def _():
                pltpu.sync_copy(acc_sh, od_hbm)

            @pl.when(c == 1)
            def _():
                pltpu.sync_copy(acc_sh, id_hbm)

    return k(src2d, dst2d, zvec)


def _sc_aggregate(m, src2d, dst2d, zrows):
    @functools.partial(
        pl.kernel,
        out_type=jax.ShapeDtypeStruct((NCORES, NN, HH), _F32),
        mesh=_vmesh(),
        scratch_types=[
            pltpu.VMEM_SHARED((NN, HH), _F32),
            pltpu.VMEM((HSPAN, CHUNK), jnp.int32),
            pltpu.VMEM((HSPAN, CHUNK), jnp.int32),
            pltpu.VMEM((CHUNK, HH), _F32),
            pltpu.VMEM((CHUNK, HH), _F32),
            pltpu.SemaphoreType.DMA,
            pltpu.SemaphoreType.DMA,
        ],
    )
    def k(m_hbm, src_hbm, dst_hbm, z_hbm, out_hbm, acc_sh, si_v, di_v,
          rows0, rows1, gsem0, gsem1):
        c = lax.axis_index("c")
        s = lax.axis_index("s")
        tid = c * NSUB + s
        j0 = tid * SPAN

        r0 = s * ROWS_PER_TILE
        pltpu.sync_copy(
            z_hbm.at[pl.ds(r0, ROWS_PER_TILE)],
            acc_sh.at[pl.ds(r0, ROWS_PER_TILE)],
        )

        @pl.when(s == NSUB - 1)
        def _():
            pltpu.sync_copy(
                z_hbm.at[pl.ds(TAIL_ROW0, TAIL_ROWS)],
                acc_sh.at[pl.ds(TAIL_ROW0, TAIL_ROWS)],
            )

        plsc.subcore_barrier()

        def _half(base):
            pltpu.sync_copy(src_hbm.at[pl.ds(base, HSPAN)], si_v)
            pltpu.sync_copy(dst_hbm.at[pl.ds(base, HSPAN)], di_v)

            @pl.when(base < NCHUNKS)
            def _():
                pltpu.async_copy(m_hbm.at[si_v.at[0]], rows0, gsem0)

            @pl.when(base + 1 < NCHUNKS)
            def _():
                pltpu.async_copy(m_hbm.at[si_v.at[1]], rows1, gsem1)

            @pl.loop(0, HSPAN, step=2)
            def _(i):
                @pl.when(base + i < NCHUNKS)
                def _():
                    pltpu.make_async_copy(
                        m_hbm.at[si_v.at[i]], rows0, gsem0).wait()
                    pltpu.sync_copy(rows0, acc_sh.at[di_v.at[i]], add=True)

                    @pl.when(jnp.logical_and(i + 2 < HSPAN,
                                             base + i + 2 < NCHUNKS))
                    def _():
                        pltpu.async_copy(m_hbm.at[si_v.at[i + 2]], rows0, gsem0)

                @pl.when(base + i + 1 < NCHUNKS)
                def _():
                    pltpu.make_async_copy(
                        m_hbm.at[si_v.at[i + 1]], rows1, gsem1).wait()
                    pltpu.sync_copy(rows1, acc_sh.at[di_v.at[i + 1]], add=True)

                    @pl.when(jnp.logical_and(i + 3 < HSPAN,
                                             base + i + 3 < NCHUNKS))
                    def _():
                        pltpu.async_copy(m_hbm.at[si_v.at[i + 3]], rows1, gsem1)

        _half(j0)
        _half(j0 + HSPAN)

        plsc.subcore_barrier()

        pltpu.sync_copy(
            acc_sh.at[pl.ds(r0, ROWS_PER_TILE)],
            out_hbm.at[c].at[pl.ds(r0, ROWS_PER_TILE)],
        )

        @pl.when(s == NSUB - 1)
        def _():
            pltpu.sync_copy(
                acc_sh.at[pl.ds(TAIL_ROW0, TAIL_ROWS)],
                out_hbm.at[c].at[pl.ds(TAIL_ROW0, TAIL_ROWS)],
            )

    return k(m, src2d, dst2d, zrows)


def _tc_norms(od, idg):
    def body(od_ref, id_ref, ns_ref, nd_ref):
        ns_ref[...] = lax.rsqrt(jnp.maximum(od_ref[...], 1.0))
        nd_ref[...] = lax.rsqrt(jnp.maximum(id_ref[...], 1.0))

    return pl.pallas_call(
        body,
        out_shape=[jax.ShapeDtypeStruct((NN,), _F32),
                   jax.ShapeDtypeStruct((NN,), _F32)],
    )(od, idg)


def _tc_layer0(x, ns, W):
    def body(x_ref, ns_ref, w_ref, o_ref):
        o_ref[...] = jnp.dot(
            x_ref[...] * ns_ref[...], w_ref[...], preferred_element_type=_F32
        )

    return pl.pallas_call(
        body,
        grid=(NBLK,),
        in_specs=[
            pl.BlockSpec((RB, HH), lambda i: (i, 0)),
            pl.BlockSpec((RB, 1), lambda i: (i, 0)),
            pl.BlockSpec((HH, HH), lambda i: (0, 0)),
        ],
        out_specs=pl.BlockSpec((RB, HH), lambda i: (i, 0)),
        out_shape=jax.ShapeDtypeStruct((NN, HH), _F32),
    )(x, ns, W)


def _tc_mid(p, nd, b, ns, W):
    def body(p_ref, nd_ref, b_ref, ns_ref, w_ref, o_ref):
        pr = p_ref[...]
        h = jnp.maximum((pr[0] + pr[1]) * nd_ref[...] + b_ref[...], 0.0)
        o_ref[...] = jnp.dot(
            h * ns_ref[...], w_ref[...], preferred_element_type=_F32
        )

    return pl.pallas_call(
        body,
        grid=(NBLK,),
        in_specs=[
            pl.BlockSpec((NCORES, RB, HH), lambda i: (0, i, 0)),
            pl.BlockSpec((RB, 1), lambda i: (i, 0)),
            pl.BlockSpec((1, HH), lambda i: (0, 0)),
            pl.BlockSpec((RB, 1), lambda i: (i, 0)),
            pl.BlockSpec((HH, HH), lambda i: (0, 0)),
        ],
        out_specs=pl.BlockSpec((RB, HH), lambda i: (i, 0)),
        out_shape=jax.ShapeDtypeStruct((NN, HH), _F32),
    )(p, nd, b, ns, W)


def _tc_final(p, nd, b, w1, b1, g, be, w2, b2):
    def body(p_ref, nd_ref, b_ref, w1_ref, b1_ref, g_ref, be_ref, w2_ref,
             b2_ref, o_ref, sacc, macc):
        i = pl.program_id(0)
        pr = p_ref[...]
        h = jnp.maximum((pr[0] + pr[1]) * nd_ref[...] + b_ref[...], 0.0)
        bs = jnp.sum(h, axis=0, keepdims=True)
        bm = jnp.max(h, axis=0, keepdims=True)

        @pl.when(i == 0)
        def _():
            sacc[...] = bs
            macc[...] = bm

        @pl.when(i > 0)
        def _():
            sacc[...] += bs
            macc[...] = jnp.maximum(macc[...], bm)

        @pl.when(i == NBLK - 1)
        def _():
            rs = sacc[...]
            rm = rs * (1.0 / NN)
            rx = macc[...]
            w1r = w1_ref[...]
            z = (
                jnp.dot(rs, w1r[0:HH], preferred_element_type=_F32)
                + jnp.dot(rm, w1r[HH:2 * HH], preferred_element_type=_F32)
                + jnp.dot(rx, w1r[2 * HH:3 * HH], preferred_element_type=_F32)
                + b1_ref[...]
            )
            z = z * (g_ref[...] * float(1.0 / np.sqrt(1.0 + EPSV))) + be_ref[...]
            z = jnp.maximum(z, 0.0)
            o_ref[...] = (
                jnp.dot(z, w2_ref[...], preferred_element_type=_F32)
                + b2_ref[...]
            )

    return pl.pallas_call(
        body,
        grid=(NBLK,),
        in_specs=[
            pl.BlockSpec((NCORES, RB, HH), lambda i: (0, i, 0)),
            pl.BlockSpec((RB, 1), lambda i: (i, 0)),
            pl.BlockSpec((1, HH), lambda i: (0, 0)),
            pl.BlockSpec((3 * HH, HH), lambda i: (0, 0)),
            pl.BlockSpec((1, HH), lambda i: (0, 0)),
            pl.BlockSpec((1, HH), lambda i: (0, 0)),
            pl.BlockSpec((1, HH), lambda i: (0, 0)),
            pl.BlockSpec((HH, 1), lambda i: (0, 0)),
            pl.BlockSpec((1, 1), lambda i: (0, 0)),
        ],
        out_specs=pl.BlockSpec((1, 1), lambda i: (0, 0)),
        out_shape=jax.ShapeDtypeStruct((1, 1), _F32),
        scratch_shapes=[
            pltpu.VMEM((1, HH), _F32),
            pltpu.VMEM((1, HH), _F32),
        ],
    )(p, nd, b, w1, b1, g, be, w2, b2)


def kernel(x, edge_index, W0, b0, W1, b1, W2, b2, W3, b3, W4, b4,
           mlpW1, mlpb1, gamma, beta, mlpW2, mlpb2):
    zvec = jnp.zeros((NN,), _F32)
    zrows = jnp.zeros((NN, HH), _F32)
    pad = jnp.zeros((NCHP * CHUNK - EE,), jnp.int32)
    src2d = jnp.concatenate([edge_index[0], pad]).reshape(NCHP, CHUNK)
    dst2d = jnp.concatenate([edge_index[1], pad]).reshape(NCHP, CHUNK)

    od, idg = _sc_degrees(src2d, dst2d, zvec)
    ns, nd = _tc_norms(od, idg)
    ns = ns.reshape(NN, 1)
    nd = nd.reshape(NN, 1)

    Ws = [W0, W1, W2, W3, W4]
    bs = [b0.reshape(1, HH), b1.reshape(1, HH), b2.reshape(1, HH),
          b3.reshape(1, HH), b4.reshape(1, HH)]

    m = _tc_layer0(x, ns, Ws[0])
    p = None
    for l in range(5):
        p = _sc_aggregate(m, src2d, dst2d, zrows)
        if l < 4:
            m = _tc_mid(p, nd, bs[l], ns, Ws[l + 1])

    return _tc_final(
        p, nd, bs[4], mlpW1, mlpb1.reshape(1, HH), gamma.reshape(1, HH),
        beta.reshape(1, HH), mlpW2, mlpb2.reshape(1, 1),
    )
